# speedup vs baseline: 6.7170x; 6.7170x over previous
import jax
import jax.numpy as jnp
from jax import lax
from jax.experimental import pallas as pl
from jax.experimental.pallas import tpu as pltpu

VOCAB_PAD = 1024
BLK = 256


def _body(x_ref, tbl_ref, out_ref, W_ref, tot_ref, pc_ref, plast_ref):
    i = pl.program_id(0)

    @pl.when(i == 0)
    def _init():
        W_ref[...] = jnp.zeros((VOCAB_PAD, 256), jnp.float32)
        tbl = tbl_ref[...]
        shifted = jnp.concatenate([tbl[:, :1], tbl[:, :127]], axis=1)
        ne = (tbl != shifted).astype(jnp.float32)
        r = lax.broadcasted_iota(jnp.int32, (128, 128), 0)
        c = lax.broadcasted_iota(jnp.int32, (128, 128), 1)
        M = (r <= c).astype(jnp.float32)
        P = jnp.dot(ne, M, preferred_element_type=jnp.float32)
        W_ref[0:1000, 0:128] = P
        W_ref[0:1000, 128:256] = tbl
        tot_ref[0] = 0
        pc_ref[0] = 0.0
        plast_ref[0] = 0.0

    xv = x_ref[...]
    iota_v = lax.broadcasted_iota(jnp.int32, (BLK, VOCAB_PAD), 1)
    oh = (xv == iota_v).astype(jnp.float32)
    G = jnp.dot(oh, W_ref[...], preferred_element_type=jnp.float32)
    Gp = G[:, 0:128]
    c_col = Gp[:, 127:128]
    f_col = G[:, 128:129]
    l_col = G[:, 255:256]

    pc = pc_ref[0]
    plv = plast_ref[0]
    c_sh = jnp.concatenate([jnp.full((1, 1), pc, jnp.float32), c_col[:-1, :]], axis=0)
    l_sh = jnp.concatenate([jnp.full((1, 1), plv, jnp.float32), l_col[:-1, :]], axis=0)
    val = c_sh + (l_sh != f_col).astype(jnp.float32)
    row = lax.broadcasted_iota(jnp.int32, (BLK, 1), 0)
    val = jnp.where((i == 0) & (row == 0), 0.0, val)

    rT = lax.broadcasted_iota(jnp.int32, (BLK, BLK), 0)
    cT = lax.broadcasted_iota(jnp.int32, (BLK, BLK), 1)
    L = (cT <= rT).astype(jnp.float32)
    S_rel = jnp.dot(L, val, preferred_element_type=jnp.float32)
    S = tot_ref[0] + S_rel.astype(jnp.int32)

    out_ref[...] = Gp.astype(jnp.int32) + S

    tot_ref[0] = tot_ref[0] + jnp.sum(val).astype(jnp.int32)
    pc_ref[0] = jnp.sum(c_col[BLK - 1:BLK, :])
    plast_ref[0] = jnp.sum(l_col[BLK - 1:BLK, :])


def kernel(x, table):
    B, Lx = x.shape
    T = B * Lx
    x2 = x.reshape(T, 1)
    grid = T // BLK
    out = pl.pallas_call(
        _body,
        grid=(grid,),
        in_specs=[
            pl.BlockSpec((BLK, 1), lambda i: (i, 0)),
            pl.BlockSpec((1000, 128), lambda i: (0, 0)),
        ],
        out_specs=pl.BlockSpec((BLK, 128), lambda i: (i, 0)),
        out_shape=jax.ShapeDtypeStruct((T, 128), jnp.int32),
        scratch_shapes=[
            pltpu.VMEM((VOCAB_PAD, 256), jnp.float32),
            pltpu.SMEM((1,), jnp.int32),
            pltpu.SMEM((1,), jnp.float32),
            pltpu.SMEM((1,), jnp.float32),
        ],
    )(x2, table)
    return out.reshape(B, Lx, 128)
